# baseline (speedup 1.0000x reference)
import functools

import jax
import jax.numpy as jnp
import numpy as np
from jax import lax
from jax.experimental import pallas as pl
from jax.experimental.pallas import tpu as pltpu
from jax.experimental.pallas import tpu_sc as plsc

N_NODES = 10000
N_EDGES = 320000
N_ENC = 128
D = 256
H = 128

NSUB = 16
EPW = N_EDGES // NSUB
C = 128
EPW_PAD = 20480
NCHUNK = EPW_PAD // C
GBLK = 16
NBLK = NCHUNK // GBLK
NACC = 10240
ROWS_PER_SUB = NACC // NSUB

RB = 1000
RGRID = N_NODES // RB
NG = 100


def _enc_body(x_ref, wn_ref, bn_ref, wlp_ref, wl_ref, vv_ref, bl_ref,
              nf_ref, h_ref, pq_ref):
    r = pl.program_id(0)
    nfb = jnp.dot(x_ref[...], wn_ref[...],
                  preferred_element_type=jnp.float32) + bn_ref[...]
    nf_ref[0] = nfb[:, :H]
    nf_ref[1] = nfb[:, H:]
    hb = jnp.dot(nfb, wlp_ref[...], preferred_element_type=jnp.float32)
    h_ref[0] = hb[:, :H]
    h_ref[1] = hb[:, H:]

    @pl.when(r == 0)
    def _():
        pq = jnp.dot(vv_ref[...], wl_ref[...],
                     preferred_element_type=jnp.float32)
        pq = pq + jnp.concatenate(
            [jnp.zeros((1, D), jnp.float32), bl_ref[...]], axis=0)
        pq_ref[0] = pq[:, :H]
        pq_ref[1] = pq[:, H:]


_TC_OUTS = [
    jax.ShapeDtypeStruct((2, N_NODES, H), jnp.float32),
    jax.ShapeDtypeStruct((2, N_NODES, H), jnp.float32),
    jax.ShapeDtypeStruct((2, 2, H), jnp.float32),
]
_TC_OUT_SPECS = [
    pl.BlockSpec((2, RB, H), lambda r: (0, r, 0)),
    pl.BlockSpec((2, RB, H), lambda r: (0, r, 0)),
    pl.BlockSpec((2, 2, H), lambda r: (0, 0, 0)),
]


def _encode(x, w_node, b_node, w_lp, w_l, vv, b_l):
    return pl.pallas_call(
        _enc_body,
        grid=(RGRID,),
        in_specs=[
            pl.BlockSpec((RB, N_ENC), lambda r: (r, 0)),
            pl.BlockSpec((N_ENC, D), lambda r: (0, 0)),
            pl.BlockSpec((1, D), lambda r: (0, 0)),
            pl.BlockSpec((D, D), lambda r: (0, 0)),
            pl.BlockSpec((D, D), lambda r: (0, 0)),
            pl.BlockSpec((2, D), lambda r: (0, 0)),
            pl.BlockSpec((1, D), lambda r: (0, 0)),
        ],
        out_specs=_TC_OUT_SPECS,
        out_shape=_TC_OUTS,
    )(x, w_node, b_node, w_lp, w_l, vv, b_l)


def _layer_body(nf_ref, agg_ref, wlp_ref, wl_ref, vv_ref, bl_ref,
                nfo_ref, h_ref, pq_ref):
    r = pl.program_id(0)
    n0 = nf_ref[0] + agg_ref[0]
    n1 = nf_ref[1] + agg_ref[1]
    nfo_ref[0] = n0
    nfo_ref[1] = n1
    nfb = jnp.concatenate([n0, n1], axis=1)
    hb = jnp.dot(nfb, wlp_ref[...], preferred_element_type=jnp.float32)
    h_ref[0] = hb[:, :H]
    h_ref[1] = hb[:, H:]

    @pl.when(r == 0)
    def _():
        pq = jnp.dot(vv_ref[...], wl_ref[...],
                     preferred_element_type=jnp.float32)
        pq = pq + jnp.concatenate(
            [jnp.zeros((1, D), jnp.float32), bl_ref[...]], axis=0)
        pq_ref[0] = pq[:, :H]
        pq_ref[1] = pq[:, H:]


def _layer_update(nf_h, agg, w_lp, w_l, vv, b_l):
    return pl.pallas_call(
        _layer_body,
        grid=(RGRID,),
        in_specs=[
            pl.BlockSpec((2, RB, H), lambda r: (0, r, 0)),
            pl.BlockSpec((2, RB, H), lambda r: (0, r, 0)),
            pl.BlockSpec((D, D), lambda r: (0, 0)),
            pl.BlockSpec((D, D), lambda r: (0, 0)),
            pl.BlockSpec((2, D), lambda r: (0, 0)),
            pl.BlockSpec((1, D), lambda r: (0, 0)),
        ],
        out_specs=_TC_OUT_SPECS,
        out_shape=_TC_OUTS,
    )(nf_h, agg, w_lp, w_l, vv, b_l)


def _readout_body(nf_ref, agg_ref, wd_ref, bd_ref, y_ref, loss_ref):
    r = pl.program_id(0)
    nfb = jnp.concatenate(
        [nf_ref[0] + agg_ref[0], nf_ref[1] + agg_ref[1]], axis=1)
    feat = jnp.sum(nfb * wd_ref[...], axis=1) + bd_ref[0, 0]
    g = jnp.mean(feat.reshape(RB // NG, NG), axis=1)
    diff = g - y_ref[0, 0, :]
    partial = jnp.sum(diff * diff)

    @pl.when(r == 0)
    def _():
        loss_ref[...] = jnp.zeros((1, 1), jnp.float32)

    loss_ref[...] = loss_ref[...] + partial

    @pl.when(r == RGRID - 1)
    def _():
        loss_ref[...] = loss_ref[...] / NG


def _readout(nf_h, agg, wdec_row, bdec, y3):
    return pl.pallas_call(
        _readout_body,
        grid=(RGRID,),
        in_specs=[
            pl.BlockSpec((2, RB, H), lambda r: (0, r, 0)),
            pl.BlockSpec((2, RB, H), lambda r: (0, r, 0)),
            pl.BlockSpec((1, D), lambda r: (0, 0)),
            pl.BlockSpec((1, 1), lambda r: (0, 0)),
            pl.BlockSpec((1, 1, RB // NG), lambda r: (r, 0, 0)),
        ],
        out_specs=pl.BlockSpec((1, 1), lambda r: (0, 0)),
        out_shape=jax.ShapeDtypeStruct((1, 1), jnp.float32),
    )(nf_h, agg, wdec_row, bdec, y3)


_MESH = plsc.VectorSubcoreMesh(core_axis_name="c", subcore_axis_name="s")


NPROW = 5120
RPS_P = NPROW // NSUB


@functools.partial(
    pl.kernel,
    mesh=_MESH,
    out_type=jax.ShapeDtypeStruct((4, NPROW, 128), jnp.float32),
    scratch_types=[
        pltpu.VMEM((GBLK, C), jnp.int32),
        pltpu.VMEM((GBLK, C), jnp.int32),
        pltpu.VMEM((GBLK, C), jnp.float32),
        pltpu.VMEM((GBLK, C), jnp.int32),
        pltpu.VMEM((GBLK, C), jnp.int32),
        pltpu.VMEM((C, 128), jnp.float32),
        pltpu.VMEM((C, 128), jnp.float32),
        pltpu.VMEM((2, 128), jnp.float32),
        pltpu.VMEM_SHARED((NPROW, 128), jnp.float32),
        pltpu.VMEM_SHARED((NPROW, 128), jnp.float32),
        pltpu.SemaphoreType.DMA,
        pltpu.SemaphoreType.DMA,
    ],
)
def _edge_pass(h_hbm, srcp_hbm, dstp_hbm, a_hbm, spar_hbm, dpar_hbm,
               pq_hbm, z_hbm, out_hbm,
               srcp_v, dstp_v, a_v, spar_v, dpar_v, g0, g1, pq_v,
               h_sp, agg_sh, semg0, semg1):
    c = lax.axis_index("c")
    s = lax.axis_index("s")
    stripe = pl.ds(s * RPS_P, RPS_P)
    zero16 = jnp.zeros((16,), jnp.float32)

    for qq in range(2):
        gq = c * 2 + qq
        pltpu.sync_copy(h_hbm.at[gq, stripe], h_sp.at[stripe])
        pltpu.sync_copy(z_hbm, agg_sh.at[stripe])
        pltpu.sync_copy(pq_hbm.at[gq], pq_v)
        plsc.subcore_barrier()

        p_chunks = [pq_v[0, pl.ds(16 * f, 16)] for f in range(4)]
        q_chunks = [pq_v[1, pl.ds(16 * f, 16)] for f in range(4)]

        def compute(gbuf, j):
            def grp_body(g_i, carry2):
                a_grp = a_v[j, pl.ds(g_i * 16, 16)]
                sp_grp = spar_v[j, pl.ds(g_i * 16, 16)]
                dp_grp = dpar_v[j, pl.ds(g_i * 16, 16)]
                for k in range(16):
                    e = g_i * 16 + k
                    a_s = a_grp[k]
                    ro = sp_grp[k] * 64
                    wo = dp_grp[k] * 64
                    nwo = 64 - wo
                    vs = [gbuf[e, pl.ds(ro + 16 * f, 16)] for f in range(4)]
                    for f in range(4):
                        gbuf[e, pl.ds(wo + 16 * f, 16)] = jnp.maximum(
                            vs[f] + a_s * p_chunks[f] + q_chunks[f], 0.0)
                    for f in range(4):
                        gbuf[e, pl.ds(nwo + 16 * f, 16)] = zero16
                return carry2

            lax.fori_loop(0, C // 16, grp_body, 0)

        def blk_body(b, carry0):
            blk = pl.ds(b * GBLK, GBLK)
            pltpu.sync_copy(srcp_hbm.at[s, blk], srcp_v)
            pltpu.sync_copy(dstp_hbm.at[s, blk], dstp_v)
            pltpu.sync_copy(a_hbm.at[s, blk], a_v)
            pltpu.sync_copy(spar_hbm.at[s, blk], spar_v)
            pltpu.sync_copy(dpar_hbm.at[s, blk], dpar_v)

            def chunk_body(j, carry):
                pltpu.async_copy(h_sp.at[srcp_v.at[j]], g0, semg0).wait()
                compute(g0, j)
                pltpu.sync_copy(g0, agg_sh.at[dstp_v.at[j]], add=True)
                return carry

            lax.fori_loop(0, GBLK, chunk_body, 0)
            return carry0

        lax.fori_loop(0, NBLK, blk_body, 0)
        plsc.subcore_barrier()
        pltpu.sync_copy(agg_sh.at[stripe], out_hbm.at[gq, stripe])


def kernel(x, edge_index, edge_attr, y, W_node, b_node, W_edge, b_edge,
           W_l0, b_l0, W_l1, b_l1, W_l2, b_l2, W_dec, b_dec):
    pad = EPW_PAD - EPW
    rs3 = (NSUB, NCHUNK, C)
    srcf = jnp.pad(edge_index[0].astype(jnp.int32).reshape(NSUB, EPW),
                   ((0, 0), (0, pad)))
    dstf = jnp.pad(edge_index[1].astype(jnp.int32).reshape(NSUB, EPW),
                   ((0, 0), (0, pad)), constant_values=N_NODES)
    a = jnp.pad(edge_attr[:, 0].reshape(NSUB, EPW),
                ((0, 0), (0, pad))).reshape(rs3)
    src_p = (srcf >> 1).reshape(rs3)
    spar = (srcf & 1).reshape(rs3)
    dst_p = (dstf >> 1).reshape(rs3)
    dpar = (dstf & 1).reshape(rs3)
    zeros = jnp.zeros((RPS_P, 128), jnp.float32)

    vv = jnp.stack([W_edge[0], b_edge])
    bn = b_node.reshape(1, D)
    y3 = y.reshape(RGRID, 1, RB // NG)
    wdec_row = W_dec.reshape(1, D)
    bdec = b_dec.reshape(1, 1)

    rpad = NPROW - N_NODES // 2

    def pack_h(h):
        h4 = h.reshape(2, N_NODES // 2, 2, 2, 64)
        quarters = [
            h4[c, :, :, q, :].reshape(N_NODES // 2, 128)
            for c in range(2) for q in range(2)
        ]
        return jnp.pad(jnp.stack(quarters), ((0, 0), (0, rpad), (0, 0)))

    def pack_pq(pq):
        zpad = jnp.zeros((2, 64), jnp.float32)
        return jnp.stack([
            jnp.concatenate([pq[c][:, 64 * q:64 * (q + 1)], zpad], axis=1)
            for c in range(2) for q in range(2)
        ])

    def unpack_agg(ap):
        return jnp.stack([
            jnp.concatenate(
                [ap[2 * c + q, :N_NODES // 2].reshape(N_NODES, 64)
                 for q in range(2)], axis=1)
            for c in range(2)
        ])

    def edge_pass(h, pq):
        ap = _edge_pass(pack_h(h), src_p, dst_p, a, spar, dpar,
                        pack_pq(pq), zeros)
        return unpack_agg(ap)

    nf, h, pq = _encode(x, W_node, bn, W_l0, W_l0, vv, b_l0.reshape(1, D))
    agg = edge_pass(h, pq)
    nf, h, pq = _layer_update(nf, agg, W_l1, W_l1, vv, b_l1.reshape(1, D))
    agg = edge_pass(h, pq)
    nf, h, pq = _layer_update(nf, agg, W_l2, W_l2, vv, b_l2.reshape(1, D))
    agg = edge_pass(h, pq)
    loss = _readout(nf, agg, wdec_row, bdec, y3)
    return jnp.reshape(loss, ())


# speedup vs baseline: 2.2412x; 2.2412x over previous
import functools

import jax
import jax.numpy as jnp
import numpy as np
from jax import lax
from jax.experimental import pallas as pl
from jax.experimental.pallas import tpu as pltpu
from jax.experimental.pallas import tpu_sc as plsc

N_NODES = 10000
N_EDGES = 320000
N_ENC = 128
D = 256
H = 128

NSUB = 16
EPW = N_EDGES // NSUB
C = 128
EPW_PAD = 20480
NCHUNK = EPW_PAD // C
GBLK = 32
NBLK = NCHUNK // GBLK
NACC = 10240
ROWS_PER_SUB = NACC // NSUB

RB = 2000
RGRID = N_NODES // RB
NG = 100


def _enc_body(x_ref, wn_ref, bn_ref, wlp_ref, wl_ref, vv_ref, bl_ref,
              nf_ref, h_ref, pq_ref):
    r = pl.program_id(0)
    nfb = jnp.dot(x_ref[...], wn_ref[...],
                  preferred_element_type=jnp.float32) + bn_ref[...]
    nf_ref[0] = nfb[:, :H]
    nf_ref[1] = nfb[:, H:]
    hb = jnp.dot(nfb, wlp_ref[...], preferred_element_type=jnp.float32)
    h_ref[0] = hb[:, :H]
    h_ref[1] = hb[:, H:]

    @pl.when(r == 0)
    def _():
        pq = jnp.dot(vv_ref[...], wl_ref[...],
                     preferred_element_type=jnp.float32)
        pq = pq + jnp.concatenate(
            [jnp.zeros((1, D), jnp.float32), bl_ref[...]], axis=0)
        pq_ref[0] = pq[:, :H]
        pq_ref[1] = pq[:, H:]


_TC_OUTS = [
    jax.ShapeDtypeStruct((2, N_NODES, H), jnp.float32),
    jax.ShapeDtypeStruct((2, N_NODES, H), jnp.float32),
    jax.ShapeDtypeStruct((2, 2, H), jnp.float32),
]
_TC_OUT_SPECS = [
    pl.BlockSpec((2, RB, H), lambda r: (0, r, 0)),
    pl.BlockSpec((2, RB, H), lambda r: (0, r, 0)),
    pl.BlockSpec((2, 2, H), lambda r: (0, 0, 0)),
]


def _encode(x, w_node, b_node, w_lp, w_l, vv, b_l):
    return pl.pallas_call(
        _enc_body,
        grid=(RGRID,),
        in_specs=[
            pl.BlockSpec((RB, N_ENC), lambda r: (r, 0)),
            pl.BlockSpec((N_ENC, D), lambda r: (0, 0)),
            pl.BlockSpec((1, D), lambda r: (0, 0)),
            pl.BlockSpec((D, D), lambda r: (0, 0)),
            pl.BlockSpec((D, D), lambda r: (0, 0)),
            pl.BlockSpec((2, D), lambda r: (0, 0)),
            pl.BlockSpec((1, D), lambda r: (0, 0)),
        ],
        out_specs=_TC_OUT_SPECS,
        out_shape=_TC_OUTS,
    )(x, w_node, b_node, w_lp, w_l, vv, b_l)


def _layer_body(nf_ref, agg_ref, wlp_ref, wl_ref, vv_ref, bl_ref,
                nfo_ref, h_ref, pq_ref):
    r = pl.program_id(0)
    n0 = nf_ref[0] + agg_ref[0]
    n1 = nf_ref[1] + agg_ref[1]
    nfo_ref[0] = n0
    nfo_ref[1] = n1
    nfb = jnp.concatenate([n0, n1], axis=1)
    hb = jnp.dot(nfb, wlp_ref[...], preferred_element_type=jnp.float32)
    h_ref[0] = hb[:, :H]
    h_ref[1] = hb[:, H:]

    @pl.when(r == 0)
    def _():
        pq = jnp.dot(vv_ref[...], wl_ref[...],
                     preferred_element_type=jnp.float32)
        pq = pq + jnp.concatenate(
            [jnp.zeros((1, D), jnp.float32), bl_ref[...]], axis=0)
        pq_ref[0] = pq[:, :H]
        pq_ref[1] = pq[:, H:]


def _layer_update(nf_h, agg, w_lp, w_l, vv, b_l):
    return pl.pallas_call(
        _layer_body,
        grid=(RGRID,),
        in_specs=[
            pl.BlockSpec((2, RB, H), lambda r: (0, r, 0)),
            pl.BlockSpec((2, RB, H), lambda r: (0, r, 0)),
            pl.BlockSpec((D, D), lambda r: (0, 0)),
            pl.BlockSpec((D, D), lambda r: (0, 0)),
            pl.BlockSpec((2, D), lambda r: (0, 0)),
            pl.BlockSpec((1, D), lambda r: (0, 0)),
        ],
        out_specs=_TC_OUT_SPECS,
        out_shape=_TC_OUTS,
    )(nf_h, agg, w_lp, w_l, vv, b_l)


def _readout_body(nf_ref, agg_ref, wd_ref, bd_ref, y_ref, loss_ref):
    r = pl.program_id(0)
    nfb = jnp.concatenate(
        [nf_ref[0] + agg_ref[0], nf_ref[1] + agg_ref[1]], axis=1)
    feat = jnp.sum(nfb * wd_ref[...], axis=1) + bd_ref[0, 0]
    g = jnp.mean(feat.reshape(RB // NG, NG), axis=1)
    diff = g - y_ref[0, 0, :]
    partial = jnp.sum(diff * diff)

    @pl.when(r == 0)
    def _():
        loss_ref[...] = jnp.zeros((1, 1), jnp.float32)

    loss_ref[...] = loss_ref[...] + partial

    @pl.when(r == RGRID - 1)
    def _():
        loss_ref[...] = loss_ref[...] / NG


def _readout(nf_h, agg, wdec_row, bdec, y3):
    return pl.pallas_call(
        _readout_body,
        grid=(RGRID,),
        in_specs=[
            pl.BlockSpec((2, RB, H), lambda r: (0, r, 0)),
            pl.BlockSpec((2, RB, H), lambda r: (0, r, 0)),
            pl.BlockSpec((1, D), lambda r: (0, 0)),
            pl.BlockSpec((1, 1), lambda r: (0, 0)),
            pl.BlockSpec((1, 1, RB // NG), lambda r: (r, 0, 0)),
        ],
        out_specs=pl.BlockSpec((1, 1), lambda r: (0, 0)),
        out_shape=jax.ShapeDtypeStruct((1, 1), jnp.float32),
    )(nf_h, agg, wdec_row, bdec, y3)


_MESH = plsc.VectorSubcoreMesh(core_axis_name="c", subcore_axis_name="s")


@functools.partial(
    pl.kernel,
    mesh=_MESH,
    out_type=jax.ShapeDtypeStruct((2, NACC, H), jnp.float32),
    scratch_types=[
        pltpu.VMEM((GBLK, C), jnp.int32),
        pltpu.VMEM((GBLK, C), jnp.int32),
        pltpu.VMEM((GBLK, C), jnp.float32),
        pltpu.VMEM((C, H), jnp.float32),
        pltpu.VMEM((C, H), jnp.float32),
        pltpu.VMEM((2, H), jnp.float32),
        pltpu.VMEM_SHARED((NACC, H), jnp.float32),
        pltpu.SemaphoreType.DMA,
        pltpu.SemaphoreType.DMA,
    ],
)
def _edge_pass(h_hbm, src_hbm, dst_hbm, a_hbm, pq_hbm, z_hbm, out_hbm,
               src_v, dst_v, a_v, g0, g1, pq_v, agg_sh, semg0, semg1):
    c = lax.axis_index("c")
    s = lax.axis_index("s")
    pltpu.sync_copy(pq_hbm.at[c], pq_v)
    pltpu.sync_copy(z_hbm, agg_sh.at[pl.ds(s * ROWS_PER_SUB, ROWS_PER_SUB)])
    plsc.subcore_barrier()

    p_chunks = [pq_v[0, pl.ds(16 * f, 16)] for f in range(H // 16)]
    q_chunks = [pq_v[1, pl.ds(16 * f, 16)] for f in range(H // 16)]
    hc = h_hbm.at[c]

    def compute(gbuf, j):
        def grp_body(g_i, carry2):
            a_grp = a_v[j, pl.ds(g_i * 16, 16)]
            for k in range(16):
                e = g_i * 16 + k
                a_s = a_grp[k]
                for f in range(H // 16):
                    r = gbuf[e, pl.ds(16 * f, 16)]
                    gbuf[e, pl.ds(16 * f, 16)] = jnp.maximum(
                        r + a_s * p_chunks[f] + q_chunks[f], 0.0)
            return carry2

        lax.fori_loop(0, C // 16, grp_body, 0)

    def blk_body(b, carry0):
        pltpu.sync_copy(src_hbm.at[s, pl.ds(b * GBLK, GBLK)], src_v)
        pltpu.sync_copy(dst_hbm.at[s, pl.ds(b * GBLK, GBLK)], dst_v)
        pltpu.sync_copy(a_hbm.at[s, pl.ds(b * GBLK, GBLK)], a_v)
        pltpu.async_copy(hc.at[src_v.at[0]], g0, semg0)

        def pair_body(m, carry):
            j0 = 2 * m
            j1 = 2 * m + 1
            pltpu.make_async_copy(hc.at[src_v.at[j0]], g0, semg0).wait()
            pltpu.async_copy(hc.at[src_v.at[j1]], g1, semg1)
            compute(g0, j0)
            pltpu.sync_copy(g0, agg_sh.at[dst_v.at[j0]], add=True)
            pltpu.make_async_copy(hc.at[src_v.at[j1]], g1, semg1).wait()

            @pl.when(m < GBLK // 2 - 1)
            def _():
                pltpu.async_copy(hc.at[src_v.at[j0 + 2]], g0, semg0)

            compute(g1, j1)
            pltpu.sync_copy(g1, agg_sh.at[dst_v.at[j1]], add=True)
            return carry

        lax.fori_loop(0, GBLK // 2, pair_body, 0)
        return carry0

    lax.fori_loop(0, NBLK, blk_body, 0)
    plsc.subcore_barrier()
    pltpu.sync_copy(
        agg_sh.at[pl.ds(s * ROWS_PER_SUB, ROWS_PER_SUB)],
        out_hbm.at[c, pl.ds(s * ROWS_PER_SUB, ROWS_PER_SUB)])


def kernel(x, edge_index, edge_attr, y, W_node, b_node, W_edge, b_edge,
           W_l0, b_l0, W_l1, b_l1, W_l2, b_l2, W_dec, b_dec):
    pad = EPW_PAD - EPW
    src = edge_index[0].astype(jnp.int32).reshape(NSUB, EPW)
    dst = edge_index[1].astype(jnp.int32).reshape(NSUB, EPW)
    a = edge_attr[:, 0].reshape(NSUB, EPW)
    src = jnp.pad(src, ((0, 0), (0, pad))).reshape(NSUB, NCHUNK, C)
    dst = jnp.pad(dst, ((0, 0), (0, pad)),
                  constant_values=N_NODES).reshape(NSUB, NCHUNK, C)
    a = jnp.pad(a, ((0, 0), (0, pad))).reshape(NSUB, NCHUNK, C)
    zeros = jnp.zeros((ROWS_PER_SUB, H), jnp.float32)

    vv = jnp.stack([W_edge[0], b_edge])
    bn = b_node.reshape(1, D)
    y3 = y.reshape(RGRID, 1, RB // NG)
    wdec_row = W_dec.reshape(1, D)
    bdec = b_dec.reshape(1, 1)

    wp0, wp1, wp2 = W_l0, W_l1, W_l2
    nf, h, pq = _encode(x, W_node, bn, wp0, W_l0, vv, b_l0.reshape(1, D))
    agg = _edge_pass(h, src, dst, a, pq, zeros)
    nf, h, pq = _layer_update(nf, agg, wp1, W_l1, vv, b_l1.reshape(1, D))
    agg = _edge_pass(h, src, dst, a, pq, zeros)
    nf, h, pq = _layer_update(nf, agg, wp2, W_l2, vv, b_l2.reshape(1, D))
    agg = _edge_pass(h, src, dst, a, pq, zeros)
    loss = _readout(nf, agg, wdec_row, bdec, y3)
    return jnp.reshape(loss, ())
